# baseline (speedup 1.0000x reference)
import functools
import jax
import jax.numpy as jnp
from jax import lax
from jax.experimental import pallas as pl
from jax.experimental.pallas import tpu as pltpu
from jax.experimental.pallas import tpu_sc as plsc

_H = 32
_W = 32
_D = 256
_B = 16
_LANES = 16


def _sc_pos_kernel(row_hbm, col_hbm, out_hbm, colbuf, rowbuf, slab, shared, sem):
    sid = lax.axis_index("s")
    cid = lax.axis_index("c")
    pltpu.sync_copy(col_hbm, colbuf)
    for rep in range(2):
        hrow = sid + rep * (_H // 2)
        pltpu.sync_copy(row_hbm.at[hrow], rowbuf)
        for c in range(_D // _LANES):
            v = rowbuf[pl.ds(c * _LANES, _LANES)]
            for i in range(_W):
                slab[i, pl.ds(c * _LANES, _LANES)] = colbuf[
                    i, pl.ds(c * _LANES, _LANES)
                ]
                slab[i, pl.ds(_D + c * _LANES, _LANES)] = v
        pltpu.sync_copy(slab, shared.at[hrow])
    plsc.subcore_barrier()
    b = cid * (_B // 2) + sid % (_B // 2)
    half = sid // (_B // 2)
    cp = pltpu.make_async_copy(
        shared.at[pl.ds(half * (_H // 2), _H // 2)],
        out_hbm.at[b, pl.ds(half * (_H // 2), _H // 2)],
        sem,
    )
    cp.start()
    cp.wait()


@functools.partial(jax.jit, static_argnums=())
def _sc_call(row_embed, col_embed):
    mesh = plsc.VectorSubcoreMesh(core_axis_name="c", subcore_axis_name="s")
    kern = functools.partial(
        pl.kernel,
        mesh=mesh,
        out_type=jax.ShapeDtypeStruct((_B, _H, _W, 2 * _D), jnp.float32),
        scratch_types=[
            pltpu.VMEM((_W, _D), jnp.float32),
            pltpu.VMEM((_D,), jnp.float32),
            pltpu.VMEM((_W, 2 * _D), jnp.float32),
            pltpu.VMEM_SHARED((_H, _W, 2 * _D), jnp.float32),
            pltpu.SemaphoreType.DMA,
        ],
    )(_sc_pos_kernel)
    return kern(row_embed, col_embed)


def kernel(x, row_embed, col_embed):
    b = x.shape[0]
    h, w = x.shape[-3], x.shape[-2]
    d = row_embed.shape[-1]
    out = _sc_call(row_embed, col_embed)
    return out.reshape(b, h * w, 2 * d)


# speedup vs baseline: 3.8645x; 3.8645x over previous
import jax
import jax.numpy as jnp
from jax.experimental import pallas as pl
from jax.experimental.pallas import tpu as pltpu

_G = 4


def _pos_kernel(row_ref, col_ref, out_hbm, g0, g1, g2, g3, sem):
    h, d = row_ref.shape
    w = col_ref.shape[0]
    b = out_hbm.shape[0]
    hg = h // _G
    col = col_ref[...]
    row = row_ref[...]
    groups = [g0, g1, g2, g3]
    copies = []
    for g, gref in enumerate(groups):
        gref[:, :, 0:d] = jnp.broadcast_to(col[None, :, :], (hg, w, d))
        gref[:, :, d : 2 * d] = jnp.broadcast_to(
            row[g * hg : (g + 1) * hg, :][:, None, :], (hg, w, d)
        )
        for i in range(b):
            c = pltpu.make_async_copy(
                gref, out_hbm.at[i, pl.ds(g * hg, hg)], sem.at[g, i]
            )
            c.start()
            copies.append(c)
    for c in copies:
        c.wait()


def kernel(x, row_embed, col_embed):
    b = x.shape[0]
    h, w = x.shape[-3], x.shape[-2]
    d = row_embed.shape[-1]
    hg = h // _G
    out = pl.pallas_call(
        _pos_kernel,
        in_specs=[
            pl.BlockSpec(memory_space=pltpu.MemorySpace.VMEM),
            pl.BlockSpec(memory_space=pltpu.MemorySpace.VMEM),
        ],
        out_specs=pl.BlockSpec(memory_space=pltpu.MemorySpace.HBM),
        out_shape=jax.ShapeDtypeStruct((b, h, w, 2 * d), row_embed.dtype),
        scratch_shapes=[
            pltpu.VMEM((hg, w, 2 * d), row_embed.dtype),
            pltpu.VMEM((hg, w, 2 * d), row_embed.dtype),
            pltpu.VMEM((hg, w, 2 * d), row_embed.dtype),
            pltpu.VMEM((hg, w, 2 * d), row_embed.dtype),
            pltpu.SemaphoreType.DMA((_G, b)),
        ],
    )(row_embed, col_embed)
    return out.reshape(b, h * w, 2 * d)


# speedup vs baseline: 3.9954x; 1.0339x over previous
import jax
import jax.numpy as jnp
from jax.experimental import pallas as pl
from jax.experimental.pallas import tpu as pltpu


def _pos_kernel(row_ref, col_ref, out_hbm, tile_ref, sem):
    h, d = row_ref.shape
    w = col_ref.shape[0]
    b = out_hbm.shape[0]
    tile_ref[:, :, 0:d] = jnp.broadcast_to(col_ref[...][None, :, :], (h, w, d))
    tile_ref[:, :, d : 2 * d] = jnp.broadcast_to(row_ref[...][:, None, :], (h, w, d))
    copies = [
        pltpu.make_async_copy(tile_ref, out_hbm.at[i], sem.at[i]) for i in range(b)
    ]
    for c in copies:
        c.start()
    for c in copies:
        c.wait()


def kernel(x, row_embed, col_embed):
    b = x.shape[0]
    h, w = x.shape[-3], x.shape[-2]
    d = row_embed.shape[-1]
    out = pl.pallas_call(
        _pos_kernel,
        in_specs=[
            pl.BlockSpec(memory_space=pltpu.MemorySpace.VMEM),
            pl.BlockSpec(memory_space=pltpu.MemorySpace.VMEM),
        ],
        out_specs=pl.BlockSpec(memory_space=pltpu.MemorySpace.HBM),
        out_shape=jax.ShapeDtypeStruct((b, h, w, 2 * d), row_embed.dtype),
        scratch_shapes=[
            pltpu.VMEM((h, w, 2 * d), row_embed.dtype),
            pltpu.SemaphoreType.DMA((b,)),
        ],
    )(row_embed, col_embed)
    return out.reshape(b, h * w, 2 * d)
